# baseline (speedup 1.0000x reference)
import jax
import jax.numpy as jnp
from jax.experimental import pallas as pl
from jax.experimental.pallas import tpu as pltpu

NUM_CANDIDATES = 1024
NUM_ENVS = 128
NUM_HORIZON = 16
ACTION_DIM = 8
NUM_TOP = 128
HD = NUM_HORIZON * ACTION_DIM
ACTION_LOW = -1.0
ACTION_HIGH = 1.0

_CAND_BLK = 64
_N_BLKS = NUM_CANDIDATES // _CAND_BLK
_GRID = 1 + 2 * _N_BLKS


def _select_mask(r_ref):
    s = r_ref[:, 0, :]
    for h in range(1, NUM_HORIZON):
        s = s + r_ref[:, h, :]

    u = jax.lax.bitcast_convert_type(s, jnp.int32)
    key = u ^ ((u >> 31) & jnp.int32(0x7FFFFFFF))

    nonneg_i = jnp.where(key >= 0, jnp.int32(1), jnp.int32(0))
    cnt_ge0 = jnp.sum(nonneg_i, axis=0, keepdims=True)
    pos_branch = cnt_ge0 >= NUM_TOP
    domain_i = jnp.where(pos_branch, nonneg_i, 1 - nonneg_i)
    kth = jnp.where(pos_branch, NUM_TOP, NUM_TOP - cnt_ge0)
    v = key & jnp.int32(0x7FFFFFFF)

    def body(i, t):
        bit = jnp.int32(1) << (jnp.int32(30) - i)
        t2 = t | bit
        c = jnp.sum(domain_i * jnp.where(v >= t2, jnp.int32(1), jnp.int32(0)),
                    axis=0, keepdims=True)
        return jnp.where(c >= kth, t2, t)

    t0 = jnp.zeros((1, NUM_ENVS), jnp.int32)
    kv = jax.lax.fori_loop(0, 31, body, t0)
    k_full = jnp.where(pos_branch, kv, kv | jnp.int32(-0x80000000))

    gt = key > k_full
    tie = key == k_full
    n_gt = jnp.sum(jnp.where(gt, jnp.int32(1), jnp.int32(0)), axis=0, keepdims=True)
    need = (NUM_TOP - n_gt).astype(jnp.float32)

    row = jax.lax.broadcasted_iota(jnp.int32, (NUM_CANDIDATES, NUM_CANDIDATES), 0)
    col = jax.lax.broadcasted_iota(jnp.int32, (NUM_CANDIDATES, NUM_CANDIDATES), 1)
    tri = jnp.where(col <= row, jnp.float32(1), jnp.float32(0))
    tie_f = jnp.where(tie, jnp.float32(1), jnp.float32(0))
    tie_rank = jax.lax.dot(tri, tie_f, precision=jax.lax.Precision.HIGHEST)

    return jnp.where(gt, 1.0, 0.0) + jnp.where(tie & (tie_rank <= need), 1.0, 0.0)


def _fused_kernel(r_ref, a_ref, e_ref, o_ref,
                  mask_ref, acc_ref, acc2_ref, mu_ref, std_ref):
    i = pl.program_id(0)

    @pl.when(i == 0)
    def _phase_select():
        mask_ref[...] = _select_mask(r_ref)
        acc_ref[...] = jnp.zeros_like(acc_ref)
        acc2_ref[...] = jnp.zeros_like(acc2_ref)

    @pl.when((i >= 1) & (i <= _N_BLKS))
    def _phase_moments():
        a = a_ref[...]
        w = mask_ref[pl.ds((i - 1) * _CAND_BLK, _CAND_BLK), :][:, None, :]
        aw = a * w
        acc_ref[...] += jnp.sum(aw, axis=0)
        acc2_ref[...] += jnp.sum(aw * a, axis=0)

    @pl.when(i == _N_BLKS)
    def _phase_finalize():
        inv = jnp.float32(1.0 / NUM_TOP)
        mu = acc_ref[...] * inv
        var = acc2_ref[...] * inv - mu * mu
        std = jnp.sqrt(jnp.maximum(var, 0.0))
        mu_ref[...] = mu
        std_ref[...] = jnp.maximum(std, 1e-6)

    @pl.when(i > _N_BLKS)
    def _phase_sample():
        o = mu_ref[...][None] + std_ref[...][None] * e_ref[...]
        o_ref[...] = jnp.clip(o, ACTION_LOW, ACTION_HIGH)


def kernel(actions, rewards):
    a3 = actions.transpose(0, 2, 3, 1).reshape(NUM_CANDIDATES, HD, NUM_ENVS)
    r3 = rewards.transpose(0, 2, 3, 1).reshape(NUM_CANDIDATES, NUM_HORIZON, NUM_ENVS)
    eps = jax.random.normal(
        jax.random.key(1),
        (NUM_CANDIDATES, NUM_ENVS, NUM_HORIZON, ACTION_DIM),
        dtype=jnp.float32,
    ).transpose(0, 2, 3, 1).reshape(NUM_CANDIDATES, HD, NUM_ENVS)

    nb = _N_BLKS

    out = pl.pallas_call(
        _fused_kernel,
        grid=(_GRID,),
        in_specs=[
            pl.BlockSpec((NUM_CANDIDATES, NUM_HORIZON, NUM_ENVS),
                         lambda i: (0, 0, 0)),
            pl.BlockSpec((_CAND_BLK, HD, NUM_ENVS),
                         lambda i: (jnp.clip(i - 1, 0, nb - 1), 0, 0)),
            pl.BlockSpec((_CAND_BLK, HD, NUM_ENVS),
                         lambda i: (jnp.maximum(i - (nb + 1), 0), 0, 0)),
        ],
        out_specs=pl.BlockSpec((_CAND_BLK, HD, NUM_ENVS),
                               lambda i: (jnp.maximum(i - (nb + 1), 0), 0, 0)),
        out_shape=jax.ShapeDtypeStruct((NUM_CANDIDATES, HD, NUM_ENVS), jnp.float32),
        scratch_shapes=[
            pltpu.VMEM((NUM_CANDIDATES, NUM_ENVS), jnp.float32),
            pltpu.VMEM((HD, NUM_ENVS), jnp.float32),
            pltpu.VMEM((HD, NUM_ENVS), jnp.float32),
            pltpu.VMEM((HD, NUM_ENVS), jnp.float32),
            pltpu.VMEM((HD, NUM_ENVS), jnp.float32),
        ],
    )(r3, a3, eps)

    return (out.reshape(NUM_CANDIDATES, NUM_HORIZON, ACTION_DIM, NUM_ENVS)
               .transpose(0, 3, 1, 2))


# speedup vs baseline: 1.0621x; 1.0621x over previous
import jax
import jax.numpy as jnp
from jax.experimental import pallas as pl
from jax.experimental.pallas import tpu as pltpu

NUM_CANDIDATES = 1024
NUM_ENVS = 128
NUM_HORIZON = 16
ACTION_DIM = 8
NUM_TOP = 128
HD = NUM_HORIZON * ACTION_DIM
ACTION_LOW = -1.0
ACTION_HIGH = 1.0

_CAND_BLK = 64
_N_BLKS = NUM_CANDIDATES // _CAND_BLK
_GRID = 1 + 2 * _N_BLKS


def _select_mask(r_ref):
    s = r_ref[:, 0, :]
    for h in range(1, NUM_HORIZON):
        s = s + r_ref[:, h, :]

    u = jax.lax.bitcast_convert_type(s, jnp.int32)
    key = u ^ ((u >> 31) & jnp.int32(0x7FFFFFFF))

    nonneg_i = jnp.where(key >= 0, jnp.int32(1), jnp.int32(0))
    cnt_ge0 = jnp.sum(nonneg_i, axis=0, keepdims=True)
    pos_branch = cnt_ge0 >= NUM_TOP
    domain_i = jnp.where(pos_branch, nonneg_i, 1 - nonneg_i)
    kth = jnp.where(pos_branch, NUM_TOP, NUM_TOP - cnt_ge0)
    v = key & jnp.int32(0x7FFFFFFF)

    def body(i, t):
        bit = jnp.int32(1) << (jnp.int32(30) - i)
        t2 = t | bit
        c = jnp.sum(domain_i * jnp.where(v >= t2, jnp.int32(1), jnp.int32(0)),
                    axis=0, keepdims=True)
        return jnp.where(c >= kth, t2, t)

    t0 = jnp.zeros((1, NUM_ENVS), jnp.int32)
    kv = jax.lax.fori_loop(0, 31, body, t0)
    k_full = jnp.where(pos_branch, kv, kv | jnp.int32(-0x80000000))

    gt = key > k_full
    tie = key == k_full
    n_gt = jnp.sum(jnp.where(gt, jnp.int32(1), jnp.int32(0)), axis=0, keepdims=True)
    need = (NUM_TOP - n_gt).astype(jnp.float32)

    row = jax.lax.broadcasted_iota(jnp.int32, (NUM_CANDIDATES, NUM_CANDIDATES), 0)
    col = jax.lax.broadcasted_iota(jnp.int32, (NUM_CANDIDATES, NUM_CANDIDATES), 1)
    tri = jnp.where(col <= row, jnp.float32(1), jnp.float32(0))
    tie_f = jnp.where(tie, jnp.float32(1), jnp.float32(0))
    tie_rank = jax.lax.dot(tri, tie_f, precision=jax.lax.Precision.HIGHEST)

    return jnp.where(gt, 1.0, 0.0) + jnp.where(tie & (tie_rank <= need), 1.0, 0.0)


def _fused_kernel(r_ref, a_ref, e_ref, o_ref,
                  mask_ref, acc_ref, acc2_ref, mu_ref, std_ref):
    i = pl.program_id(0)

    @pl.when(i == 0)
    def _phase_select():
        mask_ref[...] = _select_mask(r_ref)
        acc_ref[...] = jnp.zeros_like(acc_ref)
        acc2_ref[...] = jnp.zeros_like(acc2_ref)

    @pl.when((i >= 1) & (i <= _N_BLKS))
    def _phase_moments():
        a = a_ref[...]
        w = mask_ref[pl.ds((i - 1) * _CAND_BLK, _CAND_BLK), :][:, None, :]
        aw = a * w
        acc_ref[...] += jnp.sum(aw, axis=0)
        acc2_ref[...] += jnp.sum(aw * a, axis=0)

    @pl.when(i == _N_BLKS)
    def _phase_finalize():
        inv = jnp.float32(1.0 / NUM_TOP)
        mu = acc_ref[...] * inv
        var = acc2_ref[...] * inv - mu * mu
        std = jnp.sqrt(jnp.maximum(var, 0.0))
        mu_ref[...] = mu
        std_ref[...] = jnp.maximum(std, 1e-6)

    @pl.when(i > _N_BLKS)
    def _phase_sample():
        o = mu_ref[...][None] + std_ref[...][None] * e_ref[...]
        o_ref[...] = jnp.clip(o, ACTION_LOW, ACTION_HIGH)


def _threefry2x32(k0, k1, x0, x1):
    ks2 = k0 ^ k1 ^ jnp.uint32(0x1BD11BDA)

    def rotl(x, r):
        return (x << jnp.uint32(r)) | (x >> jnp.uint32(32 - r))

    def rounds(x0, x1, rots):
        for r in rots:
            x0 = x0 + x1
            x1 = rotl(x1, r)
            x1 = x1 ^ x0
        return x0, x1

    ra = (13, 15, 26, 6)
    rb = (17, 29, 16, 24)
    x0 = x0 + k0
    x1 = x1 + k1
    x0, x1 = rounds(x0, x1, ra)
    x0 = x0 + k1
    x1 = x1 + ks2 + jnp.uint32(1)
    x0, x1 = rounds(x0, x1, rb)
    x0 = x0 + ks2
    x1 = x1 + k0 + jnp.uint32(2)
    x0, x1 = rounds(x0, x1, ra)
    x0 = x0 + k0
    x1 = x1 + k1 + jnp.uint32(3)
    x0, x1 = rounds(x0, x1, rb)
    x0 = x0 + k1
    x1 = x1 + ks2 + jnp.uint32(4)
    x0, x1 = rounds(x0, x1, ra)
    x0 = x0 + ks2
    x1 = x1 + k0 + jnp.uint32(5)
    return x0, x1


def _eps_transposed():
    kd = jax.random.key_data(jax.random.key(1)).astype(jnp.uint32)
    k0, k1 = kd[0], kd[1]

    shape = (NUM_CANDIDATES, HD, NUM_ENVS)
    c = jax.lax.broadcasted_iota(jnp.uint32, shape, 0)
    f = jax.lax.broadcasted_iota(jnp.uint32, shape, 1)
    e = jax.lax.broadcasted_iota(jnp.uint32, shape, 2)
    p = c * jnp.uint32(NUM_ENVS * HD) + e * jnp.uint32(HD) + f
    b0, b1 = _threefry2x32(k0, k1, jnp.zeros_like(p), p)
    bits = b0 ^ b1

    fl = jax.lax.bitcast_convert_type(
        (bits >> jnp.uint32(9)) | jnp.uint32(0x3F800000), jnp.float32) - 1.0
    lo = jnp.float32(-0.9999999403953552)
    u = fl * (jnp.float32(1.0) - lo) + lo
    u = jnp.maximum(lo, u)
    return jnp.float32(1.4142135623730951) * jax.lax.erf_inv(u)


def kernel(actions, rewards):
    a3 = actions.transpose(0, 2, 3, 1).reshape(NUM_CANDIDATES, HD, NUM_ENVS)
    r3 = rewards.transpose(0, 2, 3, 1).reshape(NUM_CANDIDATES, NUM_HORIZON, NUM_ENVS)
    eps = _eps_transposed()

    nb = _N_BLKS

    out = pl.pallas_call(
        _fused_kernel,
        grid=(_GRID,),
        in_specs=[
            pl.BlockSpec((NUM_CANDIDATES, NUM_HORIZON, NUM_ENVS),
                         lambda i: (0, 0, 0)),
            pl.BlockSpec((_CAND_BLK, HD, NUM_ENVS),
                         lambda i: (jnp.clip(i - 1, 0, nb - 1), 0, 0)),
            pl.BlockSpec((_CAND_BLK, HD, NUM_ENVS),
                         lambda i: (jnp.maximum(i - (nb + 1), 0), 0, 0)),
        ],
        out_specs=pl.BlockSpec((_CAND_BLK, HD, NUM_ENVS),
                               lambda i: (jnp.maximum(i - (nb + 1), 0), 0, 0)),
        out_shape=jax.ShapeDtypeStruct((NUM_CANDIDATES, HD, NUM_ENVS), jnp.float32),
        scratch_shapes=[
            pltpu.VMEM((NUM_CANDIDATES, NUM_ENVS), jnp.float32),
            pltpu.VMEM((HD, NUM_ENVS), jnp.float32),
            pltpu.VMEM((HD, NUM_ENVS), jnp.float32),
            pltpu.VMEM((HD, NUM_ENVS), jnp.float32),
            pltpu.VMEM((HD, NUM_ENVS), jnp.float32),
        ],
    )(r3, a3, eps)

    return (out.reshape(NUM_CANDIDATES, NUM_HORIZON, ACTION_DIM, NUM_ENVS)
               .transpose(0, 3, 1, 2))
